# Initial kernel scaffold; baseline (speedup 1.0000x reference)
import jax
import jax.numpy as jnp
from jax.experimental import pallas as pl


def kernel(x, edge_index, batch, W0, b0, g0, beta0, rm0, rv0, W1, b1, g1, beta1, rm1, rv1, W2, b2, g2, beta2, rm2, rv2, Wc1, bc1, gc, betac, rmc, rvc, Wc2, bc2):
    raise NotImplementedError("write your pallas kernel here")


# speedup vs baseline: 10.7591x; 10.7591x over previous
import functools

import jax
import jax.numpy as jnp
from jax import lax
from jax.experimental import pallas as pl
from jax.experimental.pallas import tpu as pltpu
from jax.experimental.pallas import tpu_sc as plsc

N = 10000
NP = 10240
E = 320000
G = 64
D = 128
NC = 2
NS = 16
NW = NC * NS
EPW = E // NW
EC = 80
RPT = NP // NS
ZC = 128
BLK = 1024
GRID = NP // BLK

@functools.cache
def _mesh():
    return plsc.VectorSubcoreMesh(core_axis_name="c", subcore_axis_name="s",
                                  num_cores=NC, num_subcores=NS)


def _deg_body(edges_hbm, out_hbm, hist_v, idx_v):
    wid = lax.axis_index("s") * NC + lax.axis_index("c")
    zeros16 = jnp.zeros((16,), jnp.float32)
    ones16 = jnp.ones((16,), jnp.float32)

    def zbody(i, carry):
        hist_v[pl.ds(i * 16, 16)] = zeros16
        return carry
    lax.fori_loop(0, NP // 16, zbody, 0)

    base = wid * EPW
    ic = idx_v.shape[0]

    def cbody(j, carry):
        pltpu.sync_copy(edges_hbm.at[pl.ds(E + base + j * ic, ic)], idx_v)

        def ibody(k, c2):
            idx16 = idx_v[pl.ds(k * 16, 16)]
            plsc.addupdate_scatter(hist_v, [idx16], ones16)
            return c2
        lax.fori_loop(0, ic // 16, ibody, 0)
        return carry
    lax.fori_loop(0, EPW // ic, cbody, 0)

    pltpu.sync_copy(hist_v, out_hbm.at[wid])


@functools.cache
def _deg_kernel():
    return pl.kernel(
        _deg_body,
        out_type=jax.ShapeDtypeStruct((NW, NP), jnp.float32),
        mesh=_mesh(),
        compiler_params=pltpu.CompilerParams(needs_layout_passes=False),
        scratch_types=[
            pltpu.VMEM((NP,), jnp.float32),
            pltpu.VMEM((2000,), jnp.int32),
        ],
    )


def _deg_call(edge_index):
    return _deg_kernel()(edge_index)


def _scatter_body(y_hbm, edges_hbm, out_hbm, sidx_v, didx_v, rows_v, zrow_v,
                  acc_sh, sem):
    cid = lax.axis_index("c")
    sid = lax.axis_index("s")
    wid = sid * NC + cid

    zeros16 = jnp.zeros((16,), jnp.float32)

    def zbody(i, carry):
        r = i // (D // 16)
        k = i % (D // 16)
        zrow_v[r, pl.ds(k * 16, 16)] = zeros16
        return carry
    lax.fori_loop(0, ZC * (D // 16), zbody, 0)

    def zcopy(i, carry):
        pltpu.sync_copy(zrow_v, acc_sh.at[pl.ds(sid * RPT + i * ZC, ZC)])
        return carry
    lax.fori_loop(0, RPT // ZC, zcopy, 0)

    plsc.subcore_barrier()

    base = wid * EPW

    def cbody(j, carry):
        pltpu.sync_copy(edges_hbm.at[pl.ds(base + j * EC, EC)], sidx_v)
        pltpu.sync_copy(edges_hbm.at[pl.ds(E + base + j * EC, EC)], didx_v)
        pltpu.async_copy(y_hbm.at[sidx_v], rows_v, sem).wait()
        pltpu.sync_copy(rows_v, acc_sh.at[didx_v], add=True)
        return carry
    lax.fori_loop(0, EPW // EC, cbody, 0)

    plsc.subcore_barrier()

    pltpu.sync_copy(acc_sh.at[pl.ds(sid * RPT, RPT)],
                    out_hbm.at[cid, pl.ds(sid * RPT, RPT)])


@functools.cache
def _scatter_kernel():
    return pl.kernel(
        _scatter_body,
        out_type=jax.ShapeDtypeStruct((NC, NP, D), jnp.float32),
        mesh=_mesh(),
        compiler_params=pltpu.CompilerParams(needs_layout_passes=False),
        scratch_types=[
            pltpu.VMEM((EC,), jnp.int32),
            pltpu.VMEM((EC,), jnp.int32),
            pltpu.VMEM((EC, D), jnp.float32),
            pltpu.VMEM((ZC, D), jnp.float32),
            pltpu.VMEM_SHARED((NP, D), jnp.float32),
            pltpu.SemaphoreType.DMA,
        ],
    )


def _scatter_call(y, edge_index):
    return _scatter_kernel()(y, edge_index)


def _p0_body(x_ref, hist_ref, w_ref, g_ref, rv_ref, y_ref, dinv_ref):
    deg = jnp.sum(hist_ref[...], axis=0, keepdims=True) + 1.0
    dinv = lax.rsqrt(deg).reshape(BLK, 1)
    a = g_ref[...] * lax.rsqrt(rv_ref[...] + 1e-5)
    y = dinv * jnp.dot(x_ref[...], w_ref[...] * a,
                       preferred_element_type=jnp.float32)
    y_ref[...] = y
    dinv_ref[...] = dinv


def _p0(x, hist, W, g, rv):
    return pl.pallas_call(
        _p0_body,
        grid=(GRID,),
        in_specs=[
            pl.BlockSpec((BLK, D), lambda i: (i, 0)),
            pl.BlockSpec((NW, BLK), lambda i: (0, i)),
            pl.BlockSpec((D, D), lambda i: (0, 0)),
            pl.BlockSpec((1, D), lambda i: (0, 0)),
            pl.BlockSpec((1, D), lambda i: (0, 0)),
        ],
        out_specs=[
            pl.BlockSpec((BLK, D), lambda i: (i, 0)),
            pl.BlockSpec((BLK, 1), lambda i: (i, 0)),
        ],
        out_shape=[
            jax.ShapeDtypeStruct((NP, D), jnp.float32),
            jax.ShapeDtypeStruct((NP, 1), jnp.float32),
        ],
    )(x, hist, W, g, rv)


def _mid_body(use_res, acc_ref, y_ref, dinv_ref, hres_ref,
              b_ref, g_ref, beta_ref, rm_ref, rv_ref,
              wn_ref, gn_ref, rvn_ref, h_ref, yn_ref):
    a = g_ref[...] * lax.rsqrt(rv_ref[...] + 1e-5)
    bf = b_ref[...] * a + beta_ref[...] - rm_ref[...] * a
    dinv = dinv_ref[...]
    out = dinv * (acc_ref[0] + acc_ref[1] + y_ref[...]) + bf
    h = jax.nn.relu(out)
    if use_res:
        h = h + hres_ref[...]
    an = gn_ref[...] * lax.rsqrt(rvn_ref[...] + 1e-5)
    yn = dinv * jnp.dot(h, wn_ref[...] * an,
                        preferred_element_type=jnp.float32)
    h_ref[...] = h
    yn_ref[...] = yn


def _mid(use_res, acc, y, dinv, hres, b, g, beta, rm, rv, Wn, gn, rvn):
    vec = lambda: pl.BlockSpec((1, D), lambda i: (0, 0))
    return pl.pallas_call(
        functools.partial(_mid_body, use_res),
        grid=(GRID,),
        in_specs=[
            pl.BlockSpec((NC, BLK, D), lambda i: (0, i, 0)),
            pl.BlockSpec((BLK, D), lambda i: (i, 0)),
            pl.BlockSpec((BLK, 1), lambda i: (i, 0)),
            pl.BlockSpec((BLK, D), lambda i: (i, 0)),
            vec(), vec(), vec(), vec(), vec(),
            pl.BlockSpec((D, D), lambda i: (0, 0)),
            vec(), vec(),
        ],
        out_specs=[
            pl.BlockSpec((BLK, D), lambda i: (i, 0)),
            pl.BlockSpec((BLK, D), lambda i: (i, 0)),
        ],
        out_shape=[
            jax.ShapeDtypeStruct((NP, D), jnp.float32),
            jax.ShapeDtypeStruct((NP, D), jnp.float32),
        ],
    )(acc, y, dinv, hres, b, g, beta, rm, rv, Wn, gn, rvn)


def _fin_body(acc_ref, y_ref, dinv_ref, hres_ref,
              b_ref, g_ref, beta_ref, rm_ref, rv_ref,
              batch_ref, wc1_ref, bc1_ref, gc_ref, betac_ref, rmc_ref,
              rvc_ref, wc2_ref, bc2_ref, out_ref,
              sum_s, cnt_s, max_s):
    i = pl.program_id(0)

    @pl.when(i == 0)
    def _():
        sum_s[...] = jnp.zeros((G, D), jnp.float32)
        cnt_s[...] = jnp.zeros((G, D), jnp.float32)
        max_s[...] = jnp.full((G, D), -jnp.inf, jnp.float32)

    a = g_ref[...] * lax.rsqrt(rv_ref[...] + 1e-5)
    bf = b_ref[...] * a + beta_ref[...] - rm_ref[...] * a
    dinv = dinv_ref[...]
    h3 = dinv * (acc_ref[0] + acc_ref[1] + y_ref[...]) + bf + hres_ref[...]

    bb = batch_ref[...]
    seg = lax.broadcasted_iota(jnp.int32, (BLK, G), 1)
    onehot = (bb == seg).astype(jnp.float32)
    dn = (((0,), (0,)), ((), ()))
    sum_s[...] += lax.dot_general(onehot, h3, dn,
                                  preferred_element_type=jnp.float32)
    cnt_s[...] += lax.dot_general(onehot, jnp.ones((BLK, D), jnp.float32),
                                  dn, preferred_element_type=jnp.float32)

    gsel = lax.broadcasted_iota(jnp.int32, (G, 1), 0)

    def mbody(gidx, m):
        mask = bb == gidx
        cand = jnp.max(jnp.where(mask, h3, -jnp.inf), axis=0, keepdims=True)
        return jnp.where(gsel == gidx, jnp.maximum(m, cand), m)
    max_s[...] = lax.fori_loop(0, G, mbody, max_s[...])

    @pl.when(i == GRID - 1)
    def _():
        mean = sum_s[...] / jnp.maximum(cnt_s[...], 1.0)
        z = (jnp.dot(mean, wc1_ref[...][:D],
                     preferred_element_type=jnp.float32)
             + jnp.dot(max_s[...], wc1_ref[...][D:],
                       preferred_element_type=jnp.float32)
             + bc1_ref[...])
        z = jax.nn.relu(z)
        z = ((z - rmc_ref[...]) * lax.rsqrt(rvc_ref[...] + 1e-5)
             * gc_ref[...] + betac_ref[...])
        out_ref[...] = jnp.dot(z, wc2_ref[...],
                               preferred_element_type=jnp.float32) + bc2_ref[...]


def _fin(acc, y, dinv, hres, b, g, beta, rm, rv, batch2d,
         Wc1, bc1, gc, betac, rmc, rvc, Wc2, bc2):
    vec = lambda: pl.BlockSpec((1, D), lambda i: (0, 0))
    cvec = lambda: pl.BlockSpec((1, 64), lambda i: (0, 0))
    return pl.pallas_call(
        _fin_body,
        grid=(GRID,),
        in_specs=[
            pl.BlockSpec((NC, BLK, D), lambda i: (0, i, 0)),
            pl.BlockSpec((BLK, D), lambda i: (i, 0)),
            pl.BlockSpec((BLK, 1), lambda i: (i, 0)),
            pl.BlockSpec((BLK, D), lambda i: (i, 0)),
            vec(), vec(), vec(), vec(), vec(),
            pl.BlockSpec((BLK, 1), lambda i: (i, 0)),
            pl.BlockSpec((2 * D, 64), lambda i: (0, 0)),
            cvec(), cvec(), cvec(), cvec(), cvec(),
            pl.BlockSpec((64, 2), lambda i: (0, 0)),
            pl.BlockSpec((1, 2), lambda i: (0, 0)),
        ],
        out_specs=pl.BlockSpec((G, 2), lambda i: (0, 0)),
        out_shape=jax.ShapeDtypeStruct((G, 2), jnp.float32),
        scratch_shapes=[
            pltpu.VMEM((G, D), jnp.float32),
            pltpu.VMEM((G, D), jnp.float32),
            pltpu.VMEM((G, D), jnp.float32),
        ],
    )(acc, y, dinv, hres, b, g, beta, rm, rv, batch2d,
      Wc1, bc1, gc, betac, rmc, rvc, Wc2, bc2)


def kernel(x, edge_index, batch, W0, b0, g0, beta0, rm0, rv0,
           W1, b1, g1, beta1, rm1, rv1, W2, b2, g2, beta2, rm2, rv2,
           Wc1, bc1, gc, betac, rmc, rvc, Wc2, bc2):
    pad = NP - N
    xp = jnp.pad(x, ((0, pad), (0, 0)))
    batch2d = jnp.pad(batch, (0, pad), constant_values=G).reshape(NP, 1)
    row = lambda v: v.reshape(1, -1)

    eflat = edge_index.reshape(-1)
    hist = _deg_call(eflat)
    y0, dinv = _p0(xp, hist, W0, row(g0), row(rv0))
    acc0 = _scatter_call(y0, eflat)
    h1, y1 = _mid(False, acc0, y0, dinv, y0, row(b0), row(g0), row(beta0),
                  row(rm0), row(rv0), W1, row(g1), row(rv1))
    acc1 = _scatter_call(y1, eflat)
    h2, y2 = _mid(True, acc1, y1, dinv, h1, row(b1), row(g1), row(beta1),
                  row(rm1), row(rv1), W2, row(g2), row(rv2))
    acc2 = _scatter_call(y2, eflat)
    logits = _fin(acc2, y2, dinv, h2, row(b2), row(g2), row(beta2),
                  row(rm2), row(rv2), batch2d, Wc1, row(bc1), row(gc),
                  row(betac), row(rmc), row(rvc), Wc2, row(bc2))
    return logits


# speedup vs baseline: 24.3004x; 2.2586x over previous
import functools

import jax
import jax.numpy as jnp
from jax import lax
from jax.experimental import pallas as pl
from jax.experimental.pallas import tpu as pltpu
from jax.experimental.pallas import tpu_sc as plsc

N = 10000
NP = 10240
E = 320000
G = 64
D = 128
NC = 2
NS = 16
NW = NC * NS
EPW = E // NW
EC = 40
RPT = NP // NS
BLK = 1024
GRID = NP // BLK

@functools.cache
def _mesh():
    return plsc.VectorSubcoreMesh(core_axis_name="c", subcore_axis_name="s",
                                  num_cores=NC, num_subcores=NS)


def _deg_body(edges_hbm, out_hbm, hist_v, idx_v):
    wid = lax.axis_index("s") * NC + lax.axis_index("c")
    zeros16 = jnp.zeros((16,), jnp.float32)
    ones16 = jnp.ones((16,), jnp.float32)

    def zbody(i, carry):
        hist_v[pl.ds(i * 16, 16)] = zeros16
        return carry
    lax.fori_loop(0, NP // 16, zbody, 0)

    base = wid * EPW
    ic = idx_v.shape[0]

    def cbody(j, carry):
        pltpu.sync_copy(edges_hbm.at[pl.ds(E + base + j * ic, ic)], idx_v)

        def ibody(k, c2):
            idx16 = idx_v[pl.ds(k * 16, 16)]
            plsc.addupdate_scatter(hist_v, [idx16], ones16)
            return c2
        lax.fori_loop(0, ic // 16, ibody, 0)
        return carry
    lax.fori_loop(0, EPW // ic, cbody, 0)

    pltpu.sync_copy(hist_v, out_hbm.at[wid])


@functools.cache
def _deg_kernel():
    return pl.kernel(
        _deg_body,
        out_type=jax.ShapeDtypeStruct((NW, NP), jnp.float32),
        mesh=_mesh(),
        compiler_params=pltpu.CompilerParams(needs_layout_passes=False),
        scratch_types=[
            pltpu.VMEM((NP,), jnp.float32),
            pltpu.VMEM((2000,), jnp.int32),
        ],
    )


def _deg_call(edge_index):
    return _deg_kernel()(edge_index)


NBUF = 5
LOOK = 4
NCHUNK = EPW // EC


def _scatter_body(y_hbm, edges_hbm, out_hbm, sidx_v,
                  d0, d1, d2, d3, d4, r0, r1, r2, r3, r4,
                  acc_sh, sem_g, sem_i):
    didx = [d0, d1, d2, d3, d4]
    rows = [r0, r1, r2, r3, r4]
    cid = lax.axis_index("c")
    sid = lax.axis_index("s")
    wid = sid * NC + cid
    base = wid * EPW

    sld = pltpu.async_copy(edges_hbm.at[pl.ds(base, EPW)], sidx_v, sem_g)

    zeros16 = jnp.zeros((16,), jnp.float32)

    def zbody(i, carry):
        r = i // (D // 16)
        k = i % (D // 16)
        r0[r, pl.ds(k * 16, 16)] = zeros16
        return carry
    lax.fori_loop(0, EC * (D // 16), zbody, 0)

    def zcopy(i, carry):
        pltpu.sync_copy(r0, acc_sh.at[pl.ds(sid * RPT + i * EC, EC)])
        return carry
    lax.fori_loop(0, RPT // EC, zcopy, 0)

    sld.wait()

    def _issue(jc, b):
        pltpu.async_copy(edges_hbm.at[pl.ds(E + base + jc * EC, EC)],
                         didx[b], sem_i)
        pltpu.async_copy(y_hbm.at[sidx_v.at[pl.ds(jc * EC, EC)]],
                         rows[b], sem_g)

    def _proc(j, b):
        pltpu.make_async_copy(edges_hbm.at[pl.ds(base, EC)],
                              didx[b], sem_i).wait()
        pltpu.make_async_copy(y_hbm.at[sidx_v.at[pl.ds(0, EC)]],
                              rows[b], sem_g).wait()
        pltpu.sync_copy(rows[b], acc_sh.at[didx[b]], add=True)

    for b in range(LOOK):
        _issue(b, b)

    plsc.subcore_barrier()

    def gbody(g, carry):
        for b in range(NBUF):
            j = g * NBUF + b
            _proc(j, b)
            _issue(j + LOOK, (b + LOOK) % NBUF)
        return carry
    lax.fori_loop(0, NCHUNK // NBUF - 1, gbody, 0)

    for b in range(NBUF):
        j = NCHUNK - NBUF + b
        _proc(j, b)
        if j + LOOK < NCHUNK:
            _issue(j + LOOK, (b + LOOK) % NBUF)

    plsc.subcore_barrier()

    pltpu.sync_copy(acc_sh.at[pl.ds(sid * RPT, RPT)],
                    out_hbm.at[cid, pl.ds(sid * RPT, RPT)])


@functools.cache
def _scatter_kernel():
    return pl.kernel(
        _scatter_body,
        out_type=jax.ShapeDtypeStruct((NC, NP, D), jnp.float32),
        mesh=_mesh(),
        compiler_params=pltpu.CompilerParams(needs_layout_passes=False),
        scratch_types=(
            [pltpu.VMEM((EPW,), jnp.int32)]
            + [pltpu.VMEM((EC,), jnp.int32) for _ in range(NBUF)]
            + [pltpu.VMEM((EC, D), jnp.float32) for _ in range(NBUF)]
            + [
                pltpu.VMEM_SHARED((NP, D), jnp.float32),
                pltpu.SemaphoreType.DMA,
                pltpu.SemaphoreType.DMA,
            ]
        ),
    )


def _scatter_call(y, edge_index):
    return _scatter_kernel()(y, edge_index)


def _p0_body(x_ref, hist_ref, w_ref, g_ref, rv_ref, y_ref, dinv_ref):
    deg = jnp.sum(hist_ref[...], axis=0, keepdims=True) + 1.0
    dinv = lax.rsqrt(deg).reshape(BLK, 1)
    a = g_ref[...] * lax.rsqrt(rv_ref[...] + 1e-5)
    y = dinv * jnp.dot(x_ref[...], w_ref[...] * a,
                       preferred_element_type=jnp.float32)
    y_ref[...] = y
    dinv_ref[...] = dinv


def _p0(x, hist, W, g, rv):
    return pl.pallas_call(
        _p0_body,
        grid=(GRID,),
        in_specs=[
            pl.BlockSpec((BLK, D), lambda i: (i, 0)),
            pl.BlockSpec((NW, BLK), lambda i: (0, i)),
            pl.BlockSpec((D, D), lambda i: (0, 0)),
            pl.BlockSpec((1, D), lambda i: (0, 0)),
            pl.BlockSpec((1, D), lambda i: (0, 0)),
        ],
        out_specs=[
            pl.BlockSpec((BLK, D), lambda i: (i, 0)),
            pl.BlockSpec((BLK, 1), lambda i: (i, 0)),
        ],
        out_shape=[
            jax.ShapeDtypeStruct((NP, D), jnp.float32),
            jax.ShapeDtypeStruct((NP, 1), jnp.float32),
        ],
    )(x, hist, W, g, rv)


def _mid_body(use_res, acc_ref, y_ref, dinv_ref, hres_ref,
              b_ref, g_ref, beta_ref, rm_ref, rv_ref,
              wn_ref, gn_ref, rvn_ref, h_ref, yn_ref):
    a = g_ref[...] * lax.rsqrt(rv_ref[...] + 1e-5)
    bf = b_ref[...] * a + beta_ref[...] - rm_ref[...] * a
    dinv = dinv_ref[...]
    out = dinv * (acc_ref[0] + acc_ref[1] + y_ref[...]) + bf
    h = jax.nn.relu(out)
    if use_res:
        h = h + hres_ref[...]
    an = gn_ref[...] * lax.rsqrt(rvn_ref[...] + 1e-5)
    yn = dinv * jnp.dot(h, wn_ref[...] * an,
                        preferred_element_type=jnp.float32)
    h_ref[...] = h
    yn_ref[...] = yn


def _mid(use_res, acc, y, dinv, hres, b, g, beta, rm, rv, Wn, gn, rvn):
    vec = lambda: pl.BlockSpec((1, D), lambda i: (0, 0))
    return pl.pallas_call(
        functools.partial(_mid_body, use_res),
        grid=(GRID,),
        in_specs=[
            pl.BlockSpec((NC, BLK, D), lambda i: (0, i, 0)),
            pl.BlockSpec((BLK, D), lambda i: (i, 0)),
            pl.BlockSpec((BLK, 1), lambda i: (i, 0)),
            pl.BlockSpec((BLK, D), lambda i: (i, 0)),
            vec(), vec(), vec(), vec(), vec(),
            pl.BlockSpec((D, D), lambda i: (0, 0)),
            vec(), vec(),
        ],
        out_specs=[
            pl.BlockSpec((BLK, D), lambda i: (i, 0)),
            pl.BlockSpec((BLK, D), lambda i: (i, 0)),
        ],
        out_shape=[
            jax.ShapeDtypeStruct((NP, D), jnp.float32),
            jax.ShapeDtypeStruct((NP, D), jnp.float32),
        ],
    )(acc, y, dinv, hres, b, g, beta, rm, rv, Wn, gn, rvn)


def _fin_body(acc_ref, y_ref, dinv_ref, hres_ref,
              b_ref, g_ref, beta_ref, rm_ref, rv_ref,
              batch_ref, wc1_ref, bc1_ref, gc_ref, betac_ref, rmc_ref,
              rvc_ref, wc2_ref, bc2_ref, out_ref,
              sum_s, cnt_s, max_s):
    i = pl.program_id(0)

    @pl.when(i == 0)
    def _():
        sum_s[...] = jnp.zeros((G, D), jnp.float32)
        cnt_s[...] = jnp.zeros((G, D), jnp.float32)
        max_s[...] = jnp.full((G, D), -jnp.inf, jnp.float32)

    a = g_ref[...] * lax.rsqrt(rv_ref[...] + 1e-5)
    bf = b_ref[...] * a + beta_ref[...] - rm_ref[...] * a
    dinv = dinv_ref[...]
    h3 = dinv * (acc_ref[0] + acc_ref[1] + y_ref[...]) + bf + hres_ref[...]

    bb = batch_ref[...]
    seg = lax.broadcasted_iota(jnp.int32, (BLK, G), 1)
    onehot = (bb == seg).astype(jnp.float32)
    dn = (((0,), (0,)), ((), ()))
    sum_s[...] += lax.dot_general(onehot, h3, dn,
                                  preferred_element_type=jnp.float32)
    cnt_s[...] += lax.dot_general(onehot, jnp.ones((BLK, D), jnp.float32),
                                  dn, preferred_element_type=jnp.float32)

    gsel = lax.broadcasted_iota(jnp.int32, (G, 1), 0)

    def mbody(gidx, m):
        mask = bb == gidx
        cand = jnp.max(jnp.where(mask, h3, -jnp.inf), axis=0, keepdims=True)
        return jnp.where(gsel == gidx, jnp.maximum(m, cand), m)
    max_s[...] = lax.fori_loop(0, G, mbody, max_s[...])

    @pl.when(i == GRID - 1)
    def _():
        mean = sum_s[...] / jnp.maximum(cnt_s[...], 1.0)
        z = (jnp.dot(mean, wc1_ref[...][:D],
                     preferred_element_type=jnp.float32)
             + jnp.dot(max_s[...], wc1_ref[...][D:],
                       preferred_element_type=jnp.float32)
             + bc1_ref[...])
        z = jax.nn.relu(z)
        z = ((z - rmc_ref[...]) * lax.rsqrt(rvc_ref[...] + 1e-5)
             * gc_ref[...] + betac_ref[...])
        out_ref[...] = jnp.dot(z, wc2_ref[...],
                               preferred_element_type=jnp.float32) + bc2_ref[...]


def _fin(acc, y, dinv, hres, b, g, beta, rm, rv, batch2d,
         Wc1, bc1, gc, betac, rmc, rvc, Wc2, bc2):
    vec = lambda: pl.BlockSpec((1, D), lambda i: (0, 0))
    cvec = lambda: pl.BlockSpec((1, 64), lambda i: (0, 0))
    return pl.pallas_call(
        _fin_body,
        grid=(GRID,),
        in_specs=[
            pl.BlockSpec((NC, BLK, D), lambda i: (0, i, 0)),
            pl.BlockSpec((BLK, D), lambda i: (i, 0)),
            pl.BlockSpec((BLK, 1), lambda i: (i, 0)),
            pl.BlockSpec((BLK, D), lambda i: (i, 0)),
            vec(), vec(), vec(), vec(), vec(),
            pl.BlockSpec((BLK, 1), lambda i: (i, 0)),
            pl.BlockSpec((2 * D, 64), lambda i: (0, 0)),
            cvec(), cvec(), cvec(), cvec(), cvec(),
            pl.BlockSpec((64, 2), lambda i: (0, 0)),
            pl.BlockSpec((1, 2), lambda i: (0, 0)),
        ],
        out_specs=pl.BlockSpec((G, 2), lambda i: (0, 0)),
        out_shape=jax.ShapeDtypeStruct((G, 2), jnp.float32),
        scratch_shapes=[
            pltpu.VMEM((G, D), jnp.float32),
            pltpu.VMEM((G, D), jnp.float32),
            pltpu.VMEM((G, D), jnp.float32),
        ],
    )(acc, y, dinv, hres, b, g, beta, rm, rv, batch2d,
      Wc1, bc1, gc, betac, rmc, rvc, Wc2, bc2)


def kernel(x, edge_index, batch, W0, b0, g0, beta0, rm0, rv0,
           W1, b1, g1, beta1, rm1, rv1, W2, b2, g2, beta2, rm2, rv2,
           Wc1, bc1, gc, betac, rmc, rvc, Wc2, bc2):
    pad = NP - N
    xp = jnp.pad(x, ((0, pad), (0, 0)))
    batch2d = jnp.pad(batch, (0, pad), constant_values=G).reshape(NP, 1)
    row = lambda v: v.reshape(1, -1)

    eflat = edge_index.reshape(-1)
    hist = _deg_call(eflat)
    y0, dinv = _p0(xp, hist, W0, row(g0), row(rv0))
    acc0 = _scatter_call(y0, eflat)
    h1, y1 = _mid(False, acc0, y0, dinv, y0, row(b0), row(g0), row(beta0),
                  row(rm0), row(rv0), W1, row(g1), row(rv1))
    acc1 = _scatter_call(y1, eflat)
    h2, y2 = _mid(True, acc1, y1, dinv, h1, row(b1), row(g1), row(beta1),
                  row(rm1), row(rv1), W2, row(g2), row(rv2))
    acc2 = _scatter_call(y2, eflat)
    logits = _fin(acc2, y2, dinv, h2, row(b2), row(g2), row(beta2),
                  row(rm2), row(rv2), batch2d, Wc1, row(bc1), row(gc),
                  row(betac), row(rmc), row(rvc), Wc2, row(bc2))
    return logits


# speedup vs baseline: 31.9752x; 1.3158x over previous
import functools

import jax
import jax.numpy as jnp
from jax import lax
from jax.experimental import pallas as pl
from jax.experimental.pallas import tpu as pltpu
from jax.experimental.pallas import tpu_sc as plsc

N = 10000
NP = 10240
E = 320000
G = 64
D = 128
NC = 2
NS = 16
NW = NC * NS
EPW = E // NW
EC = 40
RPT = NP // NS
BLK = 1024
GRID = NP // BLK

@functools.cache
def _mesh():
    return plsc.VectorSubcoreMesh(core_axis_name="c", subcore_axis_name="s",
                                  num_cores=NC, num_subcores=NS)


def _deg_body(edges_hbm, out_hbm, hist_v, idx_v):
    wid = lax.axis_index("s") * NC + lax.axis_index("c")
    zeros16 = jnp.zeros((16,), jnp.float32)
    ones16 = jnp.ones((16,), jnp.float32)

    def zbody(i, carry):
        hist_v[pl.ds(i * 16, 16)] = zeros16
        return carry
    lax.fori_loop(0, NP // 16, zbody, 0)

    base = wid * EPW
    ic = idx_v.shape[0]

    def cbody(j, carry):
        pltpu.sync_copy(edges_hbm.at[pl.ds(E + base + j * ic, ic)], idx_v)

        def ibody(k, c2):
            idx16 = idx_v[pl.ds(k * 16, 16)]
            plsc.addupdate_scatter(hist_v, [idx16], ones16)
            return c2
        lax.fori_loop(0, ic // 16, ibody, 0)
        return carry
    lax.fori_loop(0, EPW // ic, cbody, 0)

    pltpu.sync_copy(hist_v, out_hbm.at[wid])


@functools.cache
def _deg_kernel():
    return pl.kernel(
        _deg_body,
        out_type=jax.ShapeDtypeStruct((NW, NP), jnp.float32),
        mesh=_mesh(),
        compiler_params=pltpu.CompilerParams(needs_layout_passes=False),
        scratch_types=[
            pltpu.VMEM((NP,), jnp.float32),
            pltpu.VMEM((2000,), jnp.int32),
        ],
    )


def _deg_call(edge_index):
    return _deg_kernel()(edge_index)


NBUF = 5
LOOK = 4
NCHUNK = EPW // EC


def _scatter_body(y_hbm, edges_hbm, out_hbm, sidx_v,
                  d0, d1, d2, d3, d4, r0, r1, r2, r3, r4,
                  acc_sh, sem_g, sem_i):
    didx = [d0, d1, d2, d3, d4]
    rows = [r0, r1, r2, r3, r4]
    cid = lax.axis_index("c")
    sid = lax.axis_index("s")
    wid = sid * NC + cid
    base = wid * EPW

    sld = pltpu.async_copy(edges_hbm.at[pl.ds(base, EPW)], sidx_v, sem_g)

    zeros16 = jnp.zeros((16,), jnp.float32)

    def zbody(i, carry):
        r = i // (D // 16)
        k = i % (D // 16)
        r0[r, pl.ds(k * 16, 16)] = zeros16
        return carry
    lax.fori_loop(0, EC * (D // 16), zbody, 0)

    def zcopy(i, carry):
        pltpu.async_copy(r0, acc_sh.at[pl.ds(sid * RPT + i * EC, EC)], sem_i)
        return carry
    lax.fori_loop(0, RPT // EC, zcopy, 0)

    def zwait(i, carry):
        pltpu.make_async_copy(r0, acc_sh.at[pl.ds(sid * RPT, EC)],
                              sem_i).wait()
        return carry
    lax.fori_loop(0, RPT // EC, zwait, 0)

    sld.wait()

    def _issue(jc, b):
        pltpu.async_copy(edges_hbm.at[pl.ds(E + base + jc * EC, EC)],
                         didx[b], sem_i)
        pltpu.async_copy(y_hbm.at[sidx_v.at[pl.ds(jc * EC, EC)]],
                         rows[b], sem_g)

    def _proc(j, b):
        pltpu.make_async_copy(edges_hbm.at[pl.ds(base, EC)],
                              didx[b], sem_i).wait()
        pltpu.make_async_copy(y_hbm.at[sidx_v.at[pl.ds(0, EC)]],
                              rows[b], sem_g).wait()
        pltpu.sync_copy(rows[b], acc_sh.at[didx[b]], add=True)

    for b in range(LOOK):
        _issue(b, b)

    plsc.subcore_barrier()

    def gbody(g, carry):
        for b in range(NBUF):
            j = g * NBUF + b
            _proc(j, b)
            _issue(j + LOOK, (b + LOOK) % NBUF)
        return carry
    lax.fori_loop(0, NCHUNK // NBUF - 1, gbody, 0)

    for b in range(NBUF):
        j = NCHUNK - NBUF + b
        _proc(j, b)
        if j + LOOK < NCHUNK:
            _issue(j + LOOK, (b + LOOK) % NBUF)

    plsc.subcore_barrier()

    pltpu.sync_copy(acc_sh.at[pl.ds(sid * RPT, RPT)],
                    out_hbm.at[cid, pl.ds(sid * RPT, RPT)])


@functools.cache
def _scatter_kernel():
    return pl.kernel(
        _scatter_body,
        out_type=jax.ShapeDtypeStruct((NC, NP, D), jnp.float32),
        mesh=_mesh(),
        compiler_params=pltpu.CompilerParams(needs_layout_passes=False),
        scratch_types=(
            [pltpu.VMEM((EPW,), jnp.int32)]
            + [pltpu.VMEM((EC,), jnp.int32) for _ in range(NBUF)]
            + [pltpu.VMEM((EC, D), jnp.float32) for _ in range(NBUF)]
            + [
                pltpu.VMEM_SHARED((NP, D), jnp.float32),
                pltpu.SemaphoreType.DMA,
                pltpu.SemaphoreType.DMA,
            ]
        ),
    )


def _scatter_call(y, edge_index):
    return _scatter_kernel()(y, edge_index)


def _p0_body(x_ref, hist_ref, w_ref, g_ref, rv_ref, y_ref, dinv_ref):
    deg = jnp.sum(hist_ref[...], axis=0, keepdims=True) + 1.0
    dinv = lax.rsqrt(deg).reshape(BLK, 1)
    a = g_ref[...] * lax.rsqrt(rv_ref[...] + 1e-5)
    y = dinv * jnp.dot(x_ref[...], w_ref[...] * a,
                       preferred_element_type=jnp.float32)
    y_ref[...] = y
    dinv_ref[...] = dinv


def _p0(x, hist, W, g, rv):
    return pl.pallas_call(
        _p0_body,
        grid=(GRID,),
        in_specs=[
            pl.BlockSpec((BLK, D), lambda i: (i, 0)),
            pl.BlockSpec((NW, BLK), lambda i: (0, i)),
            pl.BlockSpec((D, D), lambda i: (0, 0)),
            pl.BlockSpec((1, D), lambda i: (0, 0)),
            pl.BlockSpec((1, D), lambda i: (0, 0)),
        ],
        out_specs=[
            pl.BlockSpec((BLK, D), lambda i: (i, 0)),
            pl.BlockSpec((BLK, 1), lambda i: (i, 0)),
        ],
        out_shape=[
            jax.ShapeDtypeStruct((NP, D), jnp.float32),
            jax.ShapeDtypeStruct((NP, 1), jnp.float32),
        ],
    )(x, hist, W, g, rv)


def _mid_body(use_res, acc_ref, y_ref, dinv_ref, hres_ref,
              b_ref, g_ref, beta_ref, rm_ref, rv_ref,
              wn_ref, gn_ref, rvn_ref, h_ref, yn_ref):
    a = g_ref[...] * lax.rsqrt(rv_ref[...] + 1e-5)
    bf = b_ref[...] * a + beta_ref[...] - rm_ref[...] * a
    dinv = dinv_ref[...]
    out = dinv * (acc_ref[0] + acc_ref[1] + y_ref[...]) + bf
    h = jax.nn.relu(out)
    if use_res:
        h = h + hres_ref[...]
    an = gn_ref[...] * lax.rsqrt(rvn_ref[...] + 1e-5)
    yn = dinv * jnp.dot(h, wn_ref[...] * an,
                        preferred_element_type=jnp.float32)
    h_ref[...] = h
    yn_ref[...] = yn


def _mid(use_res, acc, y, dinv, hres, b, g, beta, rm, rv, Wn, gn, rvn):
    vec = lambda: pl.BlockSpec((1, D), lambda i: (0, 0))
    return pl.pallas_call(
        functools.partial(_mid_body, use_res),
        grid=(GRID,),
        in_specs=[
            pl.BlockSpec((NC, BLK, D), lambda i: (0, i, 0)),
            pl.BlockSpec((BLK, D), lambda i: (i, 0)),
            pl.BlockSpec((BLK, 1), lambda i: (i, 0)),
            pl.BlockSpec((BLK, D), lambda i: (i, 0)),
            vec(), vec(), vec(), vec(), vec(),
            pl.BlockSpec((D, D), lambda i: (0, 0)),
            vec(), vec(),
        ],
        out_specs=[
            pl.BlockSpec((BLK, D), lambda i: (i, 0)),
            pl.BlockSpec((BLK, D), lambda i: (i, 0)),
        ],
        out_shape=[
            jax.ShapeDtypeStruct((NP, D), jnp.float32),
            jax.ShapeDtypeStruct((NP, D), jnp.float32),
        ],
    )(acc, y, dinv, hres, b, g, beta, rm, rv, Wn, gn, rvn)


def _fin_body(acc_ref, y_ref, dinv_ref, hres_ref,
              b_ref, g_ref, beta_ref, rm_ref, rv_ref,
              batch_ref, wc1_ref, bc1_ref, gc_ref, betac_ref, rmc_ref,
              rvc_ref, wc2_ref, bc2_ref, out_ref,
              sum_s, cnt_s, max_s):
    i = pl.program_id(0)

    @pl.when(i == 0)
    def _():
        sum_s[...] = jnp.zeros((G, D), jnp.float32)
        cnt_s[...] = jnp.zeros((G, D), jnp.float32)
        max_s[...] = jnp.full((G, D), -jnp.inf, jnp.float32)

    a = g_ref[...] * lax.rsqrt(rv_ref[...] + 1e-5)
    bf = b_ref[...] * a + beta_ref[...] - rm_ref[...] * a
    dinv = dinv_ref[...]
    h3 = dinv * (acc_ref[0] + acc_ref[1] + y_ref[...]) + bf + hres_ref[...]

    bb = batch_ref[...]
    seg = lax.broadcasted_iota(jnp.int32, (BLK, G), 1)
    onehot = (bb == seg).astype(jnp.float32)
    dn = (((0,), (0,)), ((), ()))
    sum_s[...] += lax.dot_general(onehot, h3, dn,
                                  preferred_element_type=jnp.float32)
    cnt_s[...] += lax.dot_general(onehot, jnp.ones((BLK, D), jnp.float32),
                                  dn, preferred_element_type=jnp.float32)

    gsel = lax.broadcasted_iota(jnp.int32, (G, 1), 0)

    lo = bb[0, 0]
    hi = bb[BLK - 1, 0]

    def mbody(gidx, m):
        mask = bb == gidx
        cand = jnp.max(jnp.where(mask, h3, -jnp.inf), axis=0, keepdims=True)
        return jnp.where(gsel == gidx, jnp.maximum(m, cand), m)
    max_s[...] = lax.fori_loop(lo, hi + 1, mbody, max_s[...])

    @pl.when(i == GRID - 1)
    def _():
        mean = sum_s[...] / jnp.maximum(cnt_s[...], 1.0)
        z = (jnp.dot(mean, wc1_ref[...][:D],
                     preferred_element_type=jnp.float32)
             + jnp.dot(max_s[...], wc1_ref[...][D:],
                       preferred_element_type=jnp.float32)
             + bc1_ref[...])
        z = jax.nn.relu(z)
        z = ((z - rmc_ref[...]) * lax.rsqrt(rvc_ref[...] + 1e-5)
             * gc_ref[...] + betac_ref[...])
        out_ref[...] = jnp.dot(z, wc2_ref[...],
                               preferred_element_type=jnp.float32) + bc2_ref[...]


def _fin(acc, y, dinv, hres, b, g, beta, rm, rv, batch2d,
         Wc1, bc1, gc, betac, rmc, rvc, Wc2, bc2):
    vec = lambda: pl.BlockSpec((1, D), lambda i: (0, 0))
    cvec = lambda: pl.BlockSpec((1, 64), lambda i: (0, 0))
    return pl.pallas_call(
        _fin_body,
        grid=(GRID,),
        in_specs=[
            pl.BlockSpec((NC, BLK, D), lambda i: (0, i, 0)),
            pl.BlockSpec((BLK, D), lambda i: (i, 0)),
            pl.BlockSpec((BLK, 1), lambda i: (i, 0)),
            pl.BlockSpec((BLK, D), lambda i: (i, 0)),
            vec(), vec(), vec(), vec(), vec(),
            pl.BlockSpec((BLK, 1), lambda i: (i, 0)),
            pl.BlockSpec((2 * D, 64), lambda i: (0, 0)),
            cvec(), cvec(), cvec(), cvec(), cvec(),
            pl.BlockSpec((64, 2), lambda i: (0, 0)),
            pl.BlockSpec((1, 2), lambda i: (0, 0)),
        ],
        out_specs=pl.BlockSpec((G, 2), lambda i: (0, 0)),
        out_shape=jax.ShapeDtypeStruct((G, 2), jnp.float32),
        scratch_shapes=[
            pltpu.VMEM((G, D), jnp.float32),
            pltpu.VMEM((G, D), jnp.float32),
            pltpu.VMEM((G, D), jnp.float32),
        ],
    )(acc, y, dinv, hres, b, g, beta, rm, rv, batch2d,
      Wc1, bc1, gc, betac, rmc, rvc, Wc2, bc2)


def kernel(x, edge_index, batch, W0, b0, g0, beta0, rm0, rv0,
           W1, b1, g1, beta1, rm1, rv1, W2, b2, g2, beta2, rm2, rv2,
           Wc1, bc1, gc, betac, rmc, rvc, Wc2, bc2):
    pad = NP - N
    xp = jnp.pad(x, ((0, pad), (0, 0)))
    batch2d = jnp.pad(batch, (0, pad), constant_values=G).reshape(NP, 1)
    row = lambda v: v.reshape(1, -1)

    eflat = edge_index.reshape(-1)
    hist = _deg_call(eflat)
    y0, dinv = _p0(xp, hist, W0, row(g0), row(rv0))
    acc0 = _scatter_call(y0, eflat)
    h1, y1 = _mid(False, acc0, y0, dinv, y0, row(b0), row(g0), row(beta0),
                  row(rm0), row(rv0), W1, row(g1), row(rv1))
    acc1 = _scatter_call(y1, eflat)
    h2, y2 = _mid(True, acc1, y1, dinv, h1, row(b1), row(g1), row(beta1),
                  row(rm1), row(rv1), W2, row(g2), row(rv2))
    acc2 = _scatter_call(y2, eflat)
    logits = _fin(acc2, y2, dinv, h2, row(b2), row(g2), row(beta2),
                  row(rm2), row(rv2), batch2d, Wc1, row(bc1), row(gc),
                  row(betac), row(rmc), row(rvc), Wc2, row(bc2))
    return logits


# speedup vs baseline: 33.6616x; 1.0527x over previous
import functools

import jax
import jax.numpy as jnp
from jax import lax
from jax.experimental import pallas as pl
from jax.experimental.pallas import tpu as pltpu
from jax.experimental.pallas import tpu_sc as plsc

N = 10000
NP = 10240
E = 320000
G = 64
D = 128
NC = 2
NS = 16
NW = NC * NS
EPW = E // NW
EC = 40
RPT = NP // NS
BLK = 1024
GRID = NP // BLK

@functools.cache
def _mesh():
    return plsc.VectorSubcoreMesh(core_axis_name="c", subcore_axis_name="s",
                                  num_cores=NC, num_subcores=NS)


def _deg_body(edges_hbm, out_hbm, hist_v, idx_v):
    wid = lax.axis_index("s") * NC + lax.axis_index("c")
    zeros16 = jnp.zeros((16,), jnp.float32)
    ones16 = jnp.ones((16,), jnp.float32)

    def zbody(i, carry):
        hist_v[pl.ds(i * 16, 16)] = zeros16
        return carry
    lax.fori_loop(0, NP // 16, zbody, 0)

    base = wid * EPW
    ic = idx_v.shape[0]

    def cbody(j, carry):
        pltpu.sync_copy(edges_hbm.at[pl.ds(E + base + j * ic, ic)], idx_v)

        def ibody(k, c2):
            idx16 = idx_v[pl.ds(k * 16, 16)]
            plsc.addupdate_scatter(hist_v, [idx16], ones16)
            return c2
        lax.fori_loop(0, ic // 16, ibody, 0)
        return carry
    lax.fori_loop(0, EPW // ic, cbody, 0)

    pltpu.sync_copy(hist_v, out_hbm.at[wid])


@functools.cache
def _deg_kernel():
    return pl.kernel(
        _deg_body,
        out_type=jax.ShapeDtypeStruct((NW, NP), jnp.float32),
        mesh=_mesh(),
        compiler_params=pltpu.CompilerParams(needs_layout_passes=False),
        scratch_types=[
            pltpu.VMEM((NP,), jnp.float32),
            pltpu.VMEM((2000,), jnp.int32),
        ],
    )


def _deg_call(edge_index):
    return _deg_kernel()(edge_index)


NBUF = 5
LOOK = 4
NCHUNK = EPW // EC


def _scatter_body(y_hbm, edges_hbm, out_hbm, sidx_v,
                  d0, d1, d2, d3, d4, r0, r1, r2, r3, r4,
                  acc_sh, sem_g, sem_i, sem_s):
    didx = [d0, d1, d2, d3, d4]
    rows = [r0, r1, r2, r3, r4]
    cid = lax.axis_index("c")
    sid = lax.axis_index("s")
    wid = sid * NC + cid
    base = wid * EPW

    sld = pltpu.async_copy(edges_hbm.at[pl.ds(base, EPW)], sidx_v, sem_g)

    zeros16 = jnp.zeros((16,), jnp.float32)

    def zbody(i, carry):
        r = i // (D // 16)
        k = i % (D // 16)
        r0[r, pl.ds(k * 16, 16)] = zeros16
        return carry
    lax.fori_loop(0, EC * (D // 16), zbody, 0)

    def zcopy(i, carry):
        pltpu.async_copy(r0, acc_sh.at[pl.ds(sid * RPT + i * EC, EC)], sem_i)
        return carry
    lax.fori_loop(0, RPT // EC, zcopy, 0)

    def zwait(i, carry):
        pltpu.make_async_copy(r0, acc_sh.at[pl.ds(sid * RPT, EC)],
                              sem_i).wait()
        return carry
    lax.fori_loop(0, RPT // EC, zwait, 0)

    sld.wait()

    def _issue(jc, b):
        pltpu.async_copy(edges_hbm.at[pl.ds(E + base + jc * EC, EC)],
                         didx[b], sem_i)
        pltpu.async_copy(y_hbm.at[sidx_v.at[pl.ds(jc * EC, EC)]],
                         rows[b], sem_g)

    def _proc(j, b):
        pltpu.make_async_copy(edges_hbm.at[pl.ds(base, EC)],
                              didx[b], sem_i).wait()
        pltpu.make_async_copy(y_hbm.at[sidx_v.at[pl.ds(0, EC)]],
                              rows[b], sem_g).wait()
        pltpu.async_copy(rows[b], acc_sh.at[didx[b]], sem_s, add=True)

    def _swait(b):
        pltpu.make_async_copy(rows[b], acc_sh.at[didx[b]], sem_s).wait()

    for b in range(LOOK):
        _issue(b, b)

    plsc.subcore_barrier()

    _proc(0, 0)
    _issue(LOOK, LOOK % NBUF)
    for b in range(1, NBUF):
        _proc(b, b)
        _swait(b - 1)
        _issue(b + LOOK, (b + LOOK) % NBUF)

    def gbody(g, carry):
        for b in range(NBUF):
            j = g * NBUF + b
            _proc(j, b)
            _swait((b - 1) % NBUF)
            _issue(j + LOOK, (b + LOOK) % NBUF)
        return carry
    lax.fori_loop(1, NCHUNK // NBUF - 1, gbody, 0)

    for b in range(NBUF):
        j = NCHUNK - NBUF + b
        _proc(j, b)
        _swait((b - 1) % NBUF)
        if j + LOOK < NCHUNK:
            _issue(j + LOOK, (b + LOOK) % NBUF)
    _swait(NBUF - 1)

    plsc.subcore_barrier()

    pltpu.sync_copy(acc_sh.at[pl.ds(sid * RPT, RPT)],
                    out_hbm.at[cid, pl.ds(sid * RPT, RPT)])


@functools.cache
def _scatter_kernel():
    return pl.kernel(
        _scatter_body,
        out_type=jax.ShapeDtypeStruct((NC, NP, D), jnp.float32),
        mesh=_mesh(),
        compiler_params=pltpu.CompilerParams(needs_layout_passes=False),
        scratch_types=(
            [pltpu.VMEM((EPW,), jnp.int32)]
            + [pltpu.VMEM((EC,), jnp.int32) for _ in range(NBUF)]
            + [pltpu.VMEM((EC, D), jnp.float32) for _ in range(NBUF)]
            + [
                pltpu.VMEM_SHARED((NP, D), jnp.float32),
                pltpu.SemaphoreType.DMA,
                pltpu.SemaphoreType.DMA,
                pltpu.SemaphoreType.DMA,
            ]
        ),
    )


def _scatter_call(y, edge_index):
    return _scatter_kernel()(y, edge_index)


def _p0_body(x_ref, hist_ref, w_ref, g_ref, rv_ref, y_ref, dinv_ref):
    deg = jnp.sum(hist_ref[...], axis=0, keepdims=True) + 1.0
    dinv = lax.rsqrt(deg).reshape(BLK, 1)
    a = g_ref[...] * lax.rsqrt(rv_ref[...] + 1e-5)
    y = dinv * jnp.dot(x_ref[...], w_ref[...] * a,
                       preferred_element_type=jnp.float32)
    y_ref[...] = y
    dinv_ref[...] = dinv


def _p0(x, hist, W, g, rv):
    return pl.pallas_call(
        _p0_body,
        grid=(GRID,),
        in_specs=[
            pl.BlockSpec((BLK, D), lambda i: (i, 0)),
            pl.BlockSpec((NW, BLK), lambda i: (0, i)),
            pl.BlockSpec((D, D), lambda i: (0, 0)),
            pl.BlockSpec((1, D), lambda i: (0, 0)),
            pl.BlockSpec((1, D), lambda i: (0, 0)),
        ],
        out_specs=[
            pl.BlockSpec((BLK, D), lambda i: (i, 0)),
            pl.BlockSpec((BLK, 1), lambda i: (i, 0)),
        ],
        out_shape=[
            jax.ShapeDtypeStruct((NP, D), jnp.float32),
            jax.ShapeDtypeStruct((NP, 1), jnp.float32),
        ],
    )(x, hist, W, g, rv)


def _mid_body(use_res, acc_ref, y_ref, dinv_ref, hres_ref,
              b_ref, g_ref, beta_ref, rm_ref, rv_ref,
              wn_ref, gn_ref, rvn_ref, h_ref, yn_ref):
    a = g_ref[...] * lax.rsqrt(rv_ref[...] + 1e-5)
    bf = b_ref[...] * a + beta_ref[...] - rm_ref[...] * a
    dinv = dinv_ref[...]
    out = dinv * (acc_ref[0] + acc_ref[1] + y_ref[...]) + bf
    h = jax.nn.relu(out)
    if use_res:
        h = h + hres_ref[...]
    an = gn_ref[...] * lax.rsqrt(rvn_ref[...] + 1e-5)
    yn = dinv * jnp.dot(h, wn_ref[...] * an,
                        preferred_element_type=jnp.float32)
    h_ref[...] = h
    yn_ref[...] = yn


def _mid(use_res, acc, y, dinv, hres, b, g, beta, rm, rv, Wn, gn, rvn):
    vec = lambda: pl.BlockSpec((1, D), lambda i: (0, 0))
    return pl.pallas_call(
        functools.partial(_mid_body, use_res),
        grid=(GRID,),
        in_specs=[
            pl.BlockSpec((NC, BLK, D), lambda i: (0, i, 0)),
            pl.BlockSpec((BLK, D), lambda i: (i, 0)),
            pl.BlockSpec((BLK, 1), lambda i: (i, 0)),
            pl.BlockSpec((BLK, D), lambda i: (i, 0)),
            vec(), vec(), vec(), vec(), vec(),
            pl.BlockSpec((D, D), lambda i: (0, 0)),
            vec(), vec(),
        ],
        out_specs=[
            pl.BlockSpec((BLK, D), lambda i: (i, 0)),
            pl.BlockSpec((BLK, D), lambda i: (i, 0)),
        ],
        out_shape=[
            jax.ShapeDtypeStruct((NP, D), jnp.float32),
            jax.ShapeDtypeStruct((NP, D), jnp.float32),
        ],
    )(acc, y, dinv, hres, b, g, beta, rm, rv, Wn, gn, rvn)


def _fin_body(acc_ref, y_ref, dinv_ref, hres_ref,
              b_ref, g_ref, beta_ref, rm_ref, rv_ref,
              batch_ref, wc1_ref, bc1_ref, gc_ref, betac_ref, rmc_ref,
              rvc_ref, wc2_ref, bc2_ref, out_ref,
              sum_s, cnt_s, max_s):
    i = pl.program_id(0)

    @pl.when(i == 0)
    def _():
        sum_s[...] = jnp.zeros((G, D), jnp.float32)
        cnt_s[...] = jnp.zeros((G, D), jnp.float32)
        max_s[...] = jnp.full((G, D), -jnp.inf, jnp.float32)

    a = g_ref[...] * lax.rsqrt(rv_ref[...] + 1e-5)
    bf = b_ref[...] * a + beta_ref[...] - rm_ref[...] * a
    dinv = dinv_ref[...]
    h3 = dinv * (acc_ref[0] + acc_ref[1] + y_ref[...]) + bf + hres_ref[...]

    bb = batch_ref[...]
    seg = lax.broadcasted_iota(jnp.int32, (BLK, G), 1)
    onehot = (bb == seg).astype(jnp.float32)
    dn = (((0,), (0,)), ((), ()))
    sum_s[...] += lax.dot_general(onehot, h3, dn,
                                  preferred_element_type=jnp.float32)
    cnt_s[...] += lax.dot_general(onehot, jnp.ones((BLK, D), jnp.float32),
                                  dn, preferred_element_type=jnp.float32)

    gsel = lax.broadcasted_iota(jnp.int32, (G, 1), 0)

    lo = bb[0, 0]
    hi = bb[BLK - 1, 0]

    def mbody(gidx, m):
        mask = bb == gidx
        cand = jnp.max(jnp.where(mask, h3, -jnp.inf), axis=0, keepdims=True)
        return jnp.where(gsel == gidx, jnp.maximum(m, cand), m)
    max_s[...] = lax.fori_loop(lo, hi + 1, mbody, max_s[...])

    @pl.when(i == GRID - 1)
    def _():
        mean = sum_s[...] / jnp.maximum(cnt_s[...], 1.0)
        z = (jnp.dot(mean, wc1_ref[...][:D],
                     preferred_element_type=jnp.float32)
             + jnp.dot(max_s[...], wc1_ref[...][D:],
                       preferred_element_type=jnp.float32)
             + bc1_ref[...])
        z = jax.nn.relu(z)
        z = ((z - rmc_ref[...]) * lax.rsqrt(rvc_ref[...] + 1e-5)
             * gc_ref[...] + betac_ref[...])
        out_ref[...] = jnp.dot(z, wc2_ref[...],
                               preferred_element_type=jnp.float32) + bc2_ref[...]


def _fin(acc, y, dinv, hres, b, g, beta, rm, rv, batch2d,
         Wc1, bc1, gc, betac, rmc, rvc, Wc2, bc2):
    vec = lambda: pl.BlockSpec((1, D), lambda i: (0, 0))
    cvec = lambda: pl.BlockSpec((1, 64), lambda i: (0, 0))
    return pl.pallas_call(
        _fin_body,
        grid=(GRID,),
        in_specs=[
            pl.BlockSpec((NC, BLK, D), lambda i: (0, i, 0)),
            pl.BlockSpec((BLK, D), lambda i: (i, 0)),
            pl.BlockSpec((BLK, 1), lambda i: (i, 0)),
            pl.BlockSpec((BLK, D), lambda i: (i, 0)),
            vec(), vec(), vec(), vec(), vec(),
            pl.BlockSpec((BLK, 1), lambda i: (i, 0)),
            pl.BlockSpec((2 * D, 64), lambda i: (0, 0)),
            cvec(), cvec(), cvec(), cvec(), cvec(),
            pl.BlockSpec((64, 2), lambda i: (0, 0)),
            pl.BlockSpec((1, 2), lambda i: (0, 0)),
        ],
        out_specs=pl.BlockSpec((G, 2), lambda i: (0, 0)),
        out_shape=jax.ShapeDtypeStruct((G, 2), jnp.float32),
        scratch_shapes=[
            pltpu.VMEM((G, D), jnp.float32),
            pltpu.VMEM((G, D), jnp.float32),
            pltpu.VMEM((G, D), jnp.float32),
        ],
    )(acc, y, dinv, hres, b, g, beta, rm, rv, batch2d,
      Wc1, bc1, gc, betac, rmc, rvc, Wc2, bc2)


def kernel(x, edge_index, batch, W0, b0, g0, beta0, rm0, rv0,
           W1, b1, g1, beta1, rm1, rv1, W2, b2, g2, beta2, rm2, rv2,
           Wc1, bc1, gc, betac, rmc, rvc, Wc2, bc2):
    pad = NP - N
    xp = jnp.pad(x, ((0, pad), (0, 0)))
    batch2d = jnp.pad(batch, (0, pad), constant_values=G).reshape(NP, 1)
    row = lambda v: v.reshape(1, -1)

    eflat = edge_index.reshape(-1)
    hist = _deg_call(eflat)
    y0, dinv = _p0(xp, hist, W0, row(g0), row(rv0))
    acc0 = _scatter_call(y0, eflat)
    h1, y1 = _mid(False, acc0, y0, dinv, y0, row(b0), row(g0), row(beta0),
                  row(rm0), row(rv0), W1, row(g1), row(rv1))
    acc1 = _scatter_call(y1, eflat)
    h2, y2 = _mid(True, acc1, y1, dinv, h1, row(b1), row(g1), row(beta1),
                  row(rm1), row(rv1), W2, row(g2), row(rv2))
    acc2 = _scatter_call(y2, eflat)
    logits = _fin(acc2, y2, dinv, h2, row(b2), row(g2), row(beta2),
                  row(rm2), row(rv2), batch2d, Wc1, row(bc1), row(gc),
                  row(betac), row(rmc), row(rvc), Wc2, row(bc2))
    return logits
